# Initial kernel scaffold; baseline (speedup 1.0000x reference)
import functools

import jax
import jax.numpy as jnp
from jax import lax
from jax.experimental import pallas as pl
from jax.experimental.pallas import tpu as pltpu
from jax.experimental.pallas import tpu_sc as plsc

N = 10000
E = 320000
D = 128
NC = 2
NS = 16
NW = NC * NS
EPW = E // NW
K = 80
NCHUNK = EPW // K
CW = 16
RPS = N // NS


def _sc_agg_body(compute_cnt, h_hbm, src_hbm, dst_hbm, zrows_hbm, zcnt_hbm,
                 ones_hbm, agg_out, cnt_out, src_v, dst_v, rows_v, ones_v,
                 acc_sh, cnt_sh, sem):
    c = lax.axis_index("c")
    s = lax.axis_index("s")
    w = c * NS + s

    pltpu.sync_copy(zrows_hbm.at[pl.ds(s * RPS, RPS)],
                    acc_sh.at[pl.ds(s * RPS, RPS)])
    if compute_cnt:
        pltpu.sync_copy(zcnt_hbm.at[pl.ds(s * RPS, RPS)],
                        cnt_sh.at[pl.ds(s * RPS, RPS)])
        pltpu.sync_copy(ones_hbm, ones_v)
    plsc.subcore_barrier()

    def body(j, carry):
        pltpu.sync_copy(src_hbm.at[w, j], src_v)
        pltpu.sync_copy(dst_hbm.at[w, j], dst_v)
        pltpu.async_copy(h_hbm.at[src_v], rows_v, sem).wait()
        pltpu.sync_copy(rows_v, acc_sh.at[dst_v], add=True)
        if compute_cnt:
            pltpu.sync_copy(ones_v, cnt_sh.at[dst_v], add=True)
        return carry

    lax.fori_loop(0, NCHUNK, body, 0, unroll=False)
    plsc.subcore_barrier()

    pltpu.sync_copy(acc_sh.at[pl.ds(s * RPS, RPS)],
                    agg_out.at[c, pl.ds(s * RPS, RPS)])
    if compute_cnt:
        pltpu.sync_copy(cnt_sh.at[pl.ds(s * RPS, RPS)],
                        cnt_out.at[c, pl.ds(s * RPS, RPS)])


def _make_sc_agg(compute_cnt):
    scratch = [
        pltpu.VMEM((K,), jnp.int32),
        pltpu.VMEM((K,), jnp.int32),
        pltpu.VMEM((K, D), jnp.float32),
        pltpu.VMEM((K, CW), jnp.float32),
        pltpu.VMEM_SHARED((N, D), jnp.float32),
        pltpu.VMEM_SHARED((N, CW), jnp.float32),
        pltpu.SemaphoreType.DMA,
    ]
    out_type = (jax.ShapeDtypeStruct((NC, N, D), jnp.float32),
                jax.ShapeDtypeStruct((NC, N, CW), jnp.float32))
    return pl.kernel(
        functools.partial(_sc_agg_body, compute_cnt),
        out_type=out_type,
        mesh=plsc.VectorSubcoreMesh(core_axis_name="c", subcore_axis_name="s",
                                    num_cores=NC, num_subcores=NS),
        scratch_types=scratch,
        name="sc_sage_agg",
    )


_sc_agg_cnt = _make_sc_agg(True)
_sc_agg_nocnt = _make_sc_agg(False)


def _tc_body(do_relu, aggp_ref, cntp_ref, h_ref, wl_ref, bl_ref, wr_ref,
             out_ref):
    agg = aggp_ref[0] + aggp_ref[1]
    cnt = cntp_ref[0][:, 0:1] + cntp_ref[1][:, 0:1]
    mean = agg / jnp.maximum(cnt, 1.0)
    dn = (((1,), (1,)), ((), ()))
    y = (lax.dot_general(mean, wl_ref[...], dn,
                         preferred_element_type=jnp.float32)
         + lax.dot_general(h_ref[...], wr_ref[...], dn,
                           preferred_element_type=jnp.float32)
         + bl_ref[...])
    out_ref[...] = jnp.maximum(y, 0.0) if do_relu else y


def _tc_layer(aggp, cntp, h, wl, bl, wr, do_relu):
    R = 500
    grid = (N // R,)
    return pl.pallas_call(
        functools.partial(_tc_body, do_relu),
        grid=grid,
        in_specs=[
            pl.BlockSpec((NC, R, D), lambda i: (0, i, 0)),
            pl.BlockSpec((NC, R, CW), lambda i: (0, i, 0)),
            pl.BlockSpec((R, D), lambda i: (i, 0)),
            pl.BlockSpec((D, D), lambda i: (0, 0)),
            pl.BlockSpec((1, D), lambda i: (0, 0)),
            pl.BlockSpec((D, D), lambda i: (0, 0)),
        ],
        out_specs=pl.BlockSpec((R, D), lambda i: (i, 0)),
        out_shape=jax.ShapeDtypeStruct((N, D), jnp.float32),
    )(aggp, cntp, h, wl, bl.reshape(1, D), wr)


def kernel(x, edge_index, Wl1, bl1, Wr1, Wl2, bl2, Wr2, Wl3, bl3, Wr3):
    src = edge_index[0].reshape(NW, NCHUNK, K)
    dst = edge_index[1].reshape(NW, NCHUNK, K)
    zrows = jnp.zeros((N, D), jnp.float32)
    zcnt = jnp.zeros((N, CW), jnp.float32)
    ones = jnp.ones((K, CW), jnp.float32)

    aggp, cntp = _sc_agg_cnt(x, src, dst, zrows, zcnt, ones)
    h1 = _tc_layer(aggp, cntp, x, Wl1, bl1, Wr1, True)
    aggp, _ = _sc_agg_nocnt(h1, src, dst, zrows, zcnt, ones)
    h2 = _tc_layer(aggp, cntp, h1, Wl2, bl2, Wr2, True)
    aggp, _ = _sc_agg_nocnt(h2, src, dst, zrows, zcnt, ones)
    return _tc_layer(aggp, cntp, h2, Wl3, bl3, Wr3, False)


# speedup vs baseline: 4.5374x; 4.5374x over previous
import functools

import jax
import jax.numpy as jnp
from jax import lax
from jax.experimental import pallas as pl
from jax.experimental.pallas import tpu as pltpu
from jax.experimental.pallas import tpu_sc as plsc

N = 10000
E = 320000
D = 128
NC = 2
NS = 16
NW = NC * NS
EPW = E // NW
K = 80
NCHUNK = EPW // K
ZR = 1000
NZ = N // ZR


def _sc_agg_body(h_hbm, src_hbm, dst_hbm, zrows_hbm, agg_out,
                 src_v, dst_v, rows_v, acc_sh, sem):
    c = lax.axis_index("c")
    s = lax.axis_index("s")
    w = c * NS + s

    @pl.when(s < NZ)
    def _zero():
        pltpu.sync_copy(zrows_hbm.at[pl.ds(s * ZR, ZR)],
                        acc_sh.at[pl.ds(s * ZR, ZR)])

    plsc.subcore_barrier()

    def body(j, carry):
        pltpu.sync_copy(src_hbm.at[w, j], src_v)
        pltpu.sync_copy(dst_hbm.at[w, j], dst_v)
        pltpu.async_copy(h_hbm.at[src_v], rows_v, sem).wait()
        pltpu.sync_copy(rows_v, acc_sh.at[dst_v], add=True)
        return carry

    lax.fori_loop(0, NCHUNK, body, 0, unroll=False)
    plsc.subcore_barrier()

    @pl.when(s < NZ)
    def _writeout():
        pltpu.sync_copy(acc_sh.at[pl.ds(s * ZR, ZR)],
                        agg_out.at[c, pl.ds(s * ZR, ZR)])


_sc_agg = pl.kernel(
    _sc_agg_body,
    out_type=jax.ShapeDtypeStruct((NC, N, D), jnp.float32),
    mesh=plsc.VectorSubcoreMesh(core_axis_name="c", subcore_axis_name="s",
                                num_cores=NC, num_subcores=NS),
    scratch_types=[
        pltpu.VMEM((K,), jnp.int32),
        pltpu.VMEM((K,), jnp.int32),
        pltpu.VMEM((K, D), jnp.float32),
        pltpu.VMEM_SHARED((N, D), jnp.float32),
        pltpu.SemaphoreType.DMA,
    ],
    name="sc_sage_agg",
)


def _sc_cnt_body(dst_hbm, zrows_hbm, ones_hbm, cnt_out,
                 dst_v, ones_v, acc_sh, sem):
    c = lax.axis_index("c")
    s = lax.axis_index("s")
    w = c * NS + s

    @pl.when(s < NZ)
    def _zero():
        pltpu.sync_copy(zrows_hbm.at[pl.ds(s * ZR, ZR)],
                        acc_sh.at[pl.ds(s * ZR, ZR)])

    pltpu.sync_copy(ones_hbm, ones_v)
    plsc.subcore_barrier()

    def body(j, carry):
        pltpu.sync_copy(dst_hbm.at[w, j], dst_v)
        pltpu.sync_copy(ones_v, acc_sh.at[dst_v], add=True)
        return carry

    lax.fori_loop(0, NCHUNK, body, 0, unroll=False)
    plsc.subcore_barrier()

    @pl.when(s < NZ)
    def _writeout():
        pltpu.sync_copy(acc_sh.at[pl.ds(s * ZR, ZR)],
                        cnt_out.at[c, pl.ds(s * ZR, ZR)])


_sc_cnt = pl.kernel(
    _sc_cnt_body,
    out_type=jax.ShapeDtypeStruct((NC, N, D), jnp.float32),
    mesh=plsc.VectorSubcoreMesh(core_axis_name="c", subcore_axis_name="s",
                                num_cores=NC, num_subcores=NS),
    scratch_types=[
        pltpu.VMEM((K,), jnp.int32),
        pltpu.VMEM((K, D), jnp.float32),
        pltpu.VMEM_SHARED((N, D), jnp.float32),
        pltpu.SemaphoreType.DMA,
    ],
    name="sc_sage_cnt",
)


def _tc_body(do_relu, aggp_ref, cntp_ref, h_ref, wl_ref, bl_ref, wr_ref,
             out_ref):
    agg = aggp_ref[0] + aggp_ref[1]
    cnt = cntp_ref[0] + cntp_ref[1]
    mean = agg / jnp.maximum(cnt, 1.0)
    dn = (((1,), (1,)), ((), ()))
    y = (lax.dot_general(mean, wl_ref[...], dn,
                         preferred_element_type=jnp.float32)
         + lax.dot_general(h_ref[...], wr_ref[...], dn,
                           preferred_element_type=jnp.float32)
         + bl_ref[...])
    out_ref[...] = jnp.maximum(y, 0.0) if do_relu else y


def _tc_layer(aggp, cntp, h, wl, bl, wr, do_relu):
    R = 400
    grid = (N // R,)
    return pl.pallas_call(
        functools.partial(_tc_body, do_relu),
        grid=grid,
        in_specs=[
            pl.BlockSpec((NC, R, D), lambda i: (0, i, 0)),
            pl.BlockSpec((NC, R, D), lambda i: (0, i, 0)),
            pl.BlockSpec((R, D), lambda i: (i, 0)),
            pl.BlockSpec((D, D), lambda i: (0, 0)),
            pl.BlockSpec((1, D), lambda i: (0, 0)),
            pl.BlockSpec((D, D), lambda i: (0, 0)),
        ],
        out_specs=pl.BlockSpec((R, D), lambda i: (i, 0)),
        out_shape=jax.ShapeDtypeStruct((N, D), jnp.float32),
    )(aggp, cntp, h, wl, bl.reshape(1, D), wr)


def kernel(x, edge_index, Wl1, bl1, Wr1, Wl2, bl2, Wr2, Wl3, bl3, Wr3):
    src = edge_index[0].reshape(NW, NCHUNK, K)
    dst = edge_index[1].reshape(NW, NCHUNK, K)
    zrows = jnp.zeros((N, D), jnp.float32)
    ones = jnp.ones((K, D), jnp.float32)

    cntp = _sc_cnt(dst, zrows, ones)
    aggp = _sc_agg(x, src, dst, zrows)
    h1 = _tc_layer(aggp, cntp, x, Wl1, bl1, Wr1, True)
    aggp = _sc_agg(h1, src, dst, zrows)
    h2 = _tc_layer(aggp, cntp, h1, Wl2, bl2, Wr2, True)
    aggp = _sc_agg(h2, src, dst, zrows)
    return _tc_layer(aggp, cntp, h2, Wl3, bl3, Wr3, False)


# speedup vs baseline: 8.1875x; 1.8044x over previous
import functools

import jax
import jax.numpy as jnp
from jax import lax
from jax.experimental import pallas as pl
from jax.experimental.pallas import tpu as pltpu
from jax.experimental.pallas import tpu_sc as plsc

N = 10000
E = 320000
D = 128
NC = 2
NS = 16
NW = NC * NS
EPW = E // NW
K = 40
NCHUNK = EPW // K
G = 2
NR = NCHUNK // G
ZR = 1000
NZ = N // ZR


def _sc_agg_body(h_hbm, src_hbm, dst_hbm, zrows_hbm, agg_out,
                 srcv, dstv, rows, acc_sh, sem_g, sem_s, sem_is, sem_id):
    c = lax.axis_index("c")
    s = lax.axis_index("s")
    w = c * NS + s

    @pl.when(s < NZ)
    def _zero():
        pltpu.sync_copy(zrows_hbm.at[pl.ds(s * ZR, ZR)],
                        acc_sh.at[pl.ds(s * ZR, ZR)])

    plsc.subcore_barrier()

    def fire_sidx(r, p):
        for i in range(G):
            pltpu.async_copy(src_hbm.at[w, r * G + i],
                             srcv.at[p * G + i], sem_is.at[p])

    def wait_sidx(p):
        for i in range(G):
            pltpu.make_async_copy(src_hbm.at[0, 0],
                                  srcv.at[p * G + i], sem_is.at[p]).wait()

    def fire_didx(r, p):
        for i in range(G):
            pltpu.async_copy(dst_hbm.at[w, r * G + i],
                             dstv.at[p * G + i], sem_id.at[p])

    def wait_didx(p):
        for i in range(G):
            pltpu.make_async_copy(dst_hbm.at[0, 0],
                                  dstv.at[p * G + i], sem_id.at[p]).wait()

    def fire_gather(p):
        for i in range(G):
            pltpu.async_copy(
                h_hbm.at[srcv.at[p * G + i]],
                rows.at[p, i], sem_g.at[p])

    def wait_gather(p):
        for i in range(G):
            pltpu.make_async_copy(zrows_hbm.at[pl.ds(0, K)],
                                  rows.at[p, i], sem_g.at[p]).wait()

    def fire_scatter(p):
        for i in range(G):
            pltpu.async_copy(rows.at[p, i], acc_sh.at[dstv.at[p * G + i]],
                             sem_s.at[p], add=True)

    def wait_scatter(p):
        for i in range(G):
            pltpu.make_async_copy(zrows_hbm.at[pl.ds(0, K)],
                                  rows.at[p, i], sem_s.at[p]).wait()

    fire_sidx(0, 0)
    fire_sidx(1, 1)
    fire_didx(0, 0)
    wait_sidx(0)
    fire_gather(0)

    def body(r, carry):
        p = lax.rem(r, 2)
        q = 1 - p
        wait_gather(p)

        @pl.when(r > 0)
        def _drain_prev():
            wait_scatter(q)

        wait_didx(p)
        fire_scatter(p)

        @pl.when(r + 1 < NR)
        def _next():
            wait_sidx(q)
            fire_gather(q)
            fire_didx(r + 1, q)

            @pl.when(r + 2 < NR)
            def _next2():
                fire_sidx(r + 2, p)

        return carry

    lax.fori_loop(0, NR, body, 0, unroll=False)
    wait_scatter((NR - 1) % 2)
    plsc.subcore_barrier()

    @pl.when(s < NZ)
    def _writeout():
        pltpu.sync_copy(acc_sh.at[pl.ds(s * ZR, ZR)],
                        agg_out.at[c, pl.ds(s * ZR, ZR)])


_sc_agg = pl.kernel(
    _sc_agg_body,
    out_type=jax.ShapeDtypeStruct((NC, N, D), jnp.float32),
    mesh=plsc.VectorSubcoreMesh(core_axis_name="c", subcore_axis_name="s",
                                num_cores=NC, num_subcores=NS),
    scratch_types=[
        pltpu.VMEM((2 * G, K), jnp.int32),
        pltpu.VMEM((2 * G, K), jnp.int32),
        pltpu.VMEM((2, G, K, D), jnp.float32),
        pltpu.VMEM_SHARED((N, D), jnp.float32),
        pltpu.SemaphoreType.DMA((2,)),
        pltpu.SemaphoreType.DMA((2,)),
        pltpu.SemaphoreType.DMA((2,)),
        pltpu.SemaphoreType.DMA((2,)),
    ],
    name="sc_sage_agg",
)


def _sc_cnt_body(dst_hbm, zrows_hbm, ones_hbm, cnt_out,
                 dstv, ones_v, acc_sh, sem_s, sem_i):
    c = lax.axis_index("c")
    s = lax.axis_index("s")
    w = c * NS + s

    @pl.when(s < NZ)
    def _zero():
        pltpu.sync_copy(zrows_hbm.at[pl.ds(s * ZR, ZR)],
                        acc_sh.at[pl.ds(s * ZR, ZR)])

    pltpu.sync_copy(ones_hbm, ones_v)
    plsc.subcore_barrier()

    def fire_idx(r, p):
        for i in range(G):
            pltpu.async_copy(dst_hbm.at[w, r * G + i],
                             dstv.at[p * G + i], sem_i.at[p])

    def wait_idx(p):
        for i in range(G):
            pltpu.make_async_copy(dst_hbm.at[0, 0],
                                  dstv.at[p * G + i], sem_i.at[p]).wait()

    def fire_scatter(p):
        for i in range(G):
            pltpu.async_copy(ones_v, acc_sh.at[dstv.at[p * G + i]],
                             sem_s.at[p], add=True)

    def wait_scatter(p):
        for i in range(G):
            pltpu.make_async_copy(zrows_hbm.at[pl.ds(0, K)],
                                  ones_v, sem_s.at[p]).wait()

    fire_idx(0, 0)

    def body(r, carry):
        p = lax.rem(r, 2)
        q = 1 - p
        wait_idx(p)

        @pl.when(r > 0)
        def _drain_prev():
            wait_scatter(q)

        fire_scatter(p)

        @pl.when(r + 1 < NR)
        def _next():
            fire_idx(r + 1, q)

        return carry

    lax.fori_loop(0, NR, body, 0, unroll=False)
    wait_scatter((NR - 1) % 2)
    plsc.subcore_barrier()

    @pl.when(s < NZ)
    def _writeout():
        pltpu.sync_copy(acc_sh.at[pl.ds(s * ZR, ZR)],
                        cnt_out.at[c, pl.ds(s * ZR, ZR)])


_sc_cnt = pl.kernel(
    _sc_cnt_body,
    out_type=jax.ShapeDtypeStruct((NC, N, D), jnp.float32),
    mesh=plsc.VectorSubcoreMesh(core_axis_name="c", subcore_axis_name="s",
                                num_cores=NC, num_subcores=NS),
    scratch_types=[
        pltpu.VMEM((2 * G, K), jnp.int32),
        pltpu.VMEM((K, D), jnp.float32),
        pltpu.VMEM_SHARED((N, D), jnp.float32),
        pltpu.SemaphoreType.DMA((2,)),
        pltpu.SemaphoreType.DMA((2,)),
    ],
    name="sc_sage_cnt",
)


def _tc_body(do_relu, aggp_ref, cntp_ref, h_ref, wl_ref, bl_ref, wr_ref,
             out_ref):
    agg = aggp_ref[0] + aggp_ref[1]
    cnt = cntp_ref[0] + cntp_ref[1]
    mean = agg / jnp.maximum(cnt, 1.0)
    dn = (((1,), (1,)), ((), ()))
    y = (lax.dot_general(mean, wl_ref[...], dn,
                         preferred_element_type=jnp.float32)
         + lax.dot_general(h_ref[...], wr_ref[...], dn,
                           preferred_element_type=jnp.float32)
         + bl_ref[...])
    out_ref[...] = jnp.maximum(y, 0.0) if do_relu else y


def _tc_layer(aggp, cntp, h, wl, bl, wr, do_relu):
    R = 400
    grid = (N // R,)
    return pl.pallas_call(
        functools.partial(_tc_body, do_relu),
        grid=grid,
        in_specs=[
            pl.BlockSpec((NC, R, D), lambda i: (0, i, 0)),
            pl.BlockSpec((NC, R, D), lambda i: (0, i, 0)),
            pl.BlockSpec((R, D), lambda i: (i, 0)),
            pl.BlockSpec((D, D), lambda i: (0, 0)),
            pl.BlockSpec((1, D), lambda i: (0, 0)),
            pl.BlockSpec((D, D), lambda i: (0, 0)),
        ],
        out_specs=pl.BlockSpec((R, D), lambda i: (i, 0)),
        out_shape=jax.ShapeDtypeStruct((N, D), jnp.float32),
    )(aggp, cntp, h, wl, bl.reshape(1, D), wr)


def kernel(x, edge_index, Wl1, bl1, Wr1, Wl2, bl2, Wr2, Wl3, bl3, Wr3):
    src = edge_index[0].reshape(NW, NCHUNK, K)
    dst = edge_index[1].reshape(NW, NCHUNK, K)
    zrows = jnp.zeros((N, D), jnp.float32)
    ones = jnp.ones((K, D), jnp.float32)

    cntp = _sc_cnt(dst, zrows, ones)
    aggp = _sc_agg(x, src, dst, zrows)
    h1 = _tc_layer(aggp, cntp, x, Wl1, bl1, Wr1, True)
    aggp = _sc_agg(h1, src, dst, zrows)
    h2 = _tc_layer(aggp, cntp, h1, Wl2, bl2, Wr2, True)
    aggp = _sc_agg(h2, src, dst, zrows)
    return _tc_layer(aggp, cntp, h2, Wl3, bl3, Wr3, False)
